# Initial kernel scaffold; baseline (speedup 1.0000x reference)
import jax
import jax.numpy as jnp
from jax.experimental import pallas as pl


def kernel(op_feats, config_feats, emb_table, op_weights, config_weights, W, b, op_code):
    raise NotImplementedError("write your pallas kernel here")


# speedup vs baseline: 1.3633x; 1.3633x over previous
import jax
import jax.numpy as jnp
from jax.experimental import pallas as pl
from jax.experimental.pallas import tpu as pltpu

_OPF = 140
_EMB = 128
_CFG = 18
_OUT = 128


def _prep_kernel(emb_ref, wemb_ref, wcfg_ref, cfgwt_ref, opw_ref, b_ref,
                 proj_ref, wcfgf_ref):
    emb = emb_ref[...]
    norm = jnp.sqrt(jnp.sum(emb * emb, axis=1, keepdims=True))
    scale = jnp.minimum(1.0, 1.0 / jnp.maximum(norm, 1e-7)) * opw_ref[0, 0]
    proj_ref[...] = (
        jnp.dot(emb * scale, wemb_ref[...], preferred_element_type=jnp.float32)
        + b_ref[...]
    )
    wcfgf_ref[...] = wcfg_ref[...] * cfgwt_ref[...]


def _main_kernel(opf_ref, cfg_ref, idx_ref, wop_ref, wcfgf_ref, proj_ref,
                 out_ref):
    blk = opf_ref.shape[0]
    onehot = (idx_ref[...] ==
              jax.lax.broadcasted_iota(jnp.int32, (blk, _EMB), 1)
              ).astype(jnp.float32)
    acc = jnp.dot(opf_ref[...], wop_ref[...],
                  preferred_element_type=jnp.float32)
    acc += jnp.dot(cfg_ref[...], wcfgf_ref[...],
                   preferred_element_type=jnp.float32)
    acc += jnp.dot(onehot, proj_ref[...], preferred_element_type=jnp.float32)
    out_ref[...] = acc


def kernel(op_feats, config_feats, emb_table, op_weights, config_weights,
           W, b, op_code):
    n = op_feats.shape[0]
    w_op = W[0:_OPF]
    w_emb = W[_OPF:_OPF + _EMB]
    w_cfg = W[_OPF + _EMB:]
    cfgw_t = config_weights.reshape(_CFG, 1)
    b2 = b.reshape(1, _OUT)

    emb_proj, w_cfg_f = pl.pallas_call(
        _prep_kernel,
        out_shape=[
            jax.ShapeDtypeStruct((_EMB, _OUT), jnp.float32),
            jax.ShapeDtypeStruct((_CFG, _OUT), jnp.float32),
        ],
    )(emb_table, w_emb, w_cfg, cfgw_t, op_weights, b2)

    idx = op_code.astype(jnp.int32).reshape(n, 1)

    blk = 2000
    grid = n // blk
    assert grid * blk == n

    out = pl.pallas_call(
        _main_kernel,
        grid=(grid,),
        in_specs=[
            pl.BlockSpec((blk, _OPF), lambda i: (i, 0)),
            pl.BlockSpec((blk, _CFG), lambda i: (i, 0)),
            pl.BlockSpec((blk, 1), lambda i: (i, 0)),
            pl.BlockSpec((_OPF, _OUT), lambda i: (0, 0)),
            pl.BlockSpec((_CFG, _OUT), lambda i: (0, 0)),
            pl.BlockSpec((_EMB, _OUT), lambda i: (0, 0)),
        ],
        out_specs=pl.BlockSpec((blk, _OUT), lambda i: (i, 0)),
        out_shape=jax.ShapeDtypeStruct((n, _OUT), jnp.float32),
        compiler_params=pltpu.CompilerParams(
            dimension_semantics=("parallel",)),
    )(op_feats, config_feats, idx, w_op, w_cfg_f, emb_proj)
    return out


# speedup vs baseline: 1.7125x; 1.2562x over previous
import jax
import jax.numpy as jnp
from jax.experimental import pallas as pl
from jax.experimental.pallas import tpu as pltpu

_OPF = 140
_EMB = 128
_CFG = 18
_OUT = 128


def _prep_kernel(emb_ref, wemb_ref, wcfg_ref, cfgwt_ref, opw_ref, b_ref,
                 proj_ref, wcfgf_ref):
    emb = emb_ref[...]
    norm = jnp.sqrt(jnp.sum(emb * emb, axis=1, keepdims=True))
    scale = jnp.minimum(1.0, 1.0 / jnp.maximum(norm, 1e-7)) * opw_ref[0, 0]
    proj_ref[...] = (
        jnp.dot(emb * scale, wemb_ref[...], preferred_element_type=jnp.float32)
        + b_ref[...]
    )
    wcfgf_ref[...] = wcfg_ref[...] * cfgwt_ref[...]


def _main_kernel(opf_ref, cfg_ref, idx_ref, wop_ref, wcfgf_ref, proj_ref,
                 out_ref):
    blk = opf_ref.shape[0]
    idx = idx_ref[...].reshape(1, blk)
    oh_t = (jax.lax.broadcasted_iota(jnp.int32, (_EMB, blk), 0) == idx
            ).astype(jnp.float32)
    acc = jnp.dot(opf_ref[...], wop_ref[...],
                  preferred_element_type=jnp.float32)
    acc += jnp.dot(cfg_ref[...], wcfgf_ref[...],
                   preferred_element_type=jnp.float32)
    acc += jax.lax.dot_general(oh_t, proj_ref[...],
                               (((0,), (0,)), ((), ())),
                               preferred_element_type=jnp.float32)
    out_ref[...] = acc


def kernel(op_feats, config_feats, emb_table, op_weights, config_weights,
           W, b, op_code):
    n = op_feats.shape[0]
    w_op = W[0:_OPF]
    w_emb = W[_OPF:_OPF + _EMB]
    w_cfg = W[_OPF + _EMB:]
    cfgw_t = config_weights.reshape(_CFG, 1)
    b2 = b.reshape(1, _OUT)

    emb_proj, w_cfg_f = pl.pallas_call(
        _prep_kernel,
        out_shape=[
            jax.ShapeDtypeStruct((_EMB, _OUT), jnp.float32),
            jax.ShapeDtypeStruct((_CFG, _OUT), jnp.float32),
        ],
    )(emb_table, w_emb, w_cfg, cfgw_t, op_weights, b2)

    blk = 2000
    grid = n // blk
    assert grid * blk == n
    idx = op_code.astype(jnp.int32).reshape(grid, 1, blk)

    out = pl.pallas_call(
        _main_kernel,
        grid=(grid,),
        in_specs=[
            pl.BlockSpec((blk, _OPF), lambda i: (i, 0)),
            pl.BlockSpec((blk, _CFG), lambda i: (i, 0)),
            pl.BlockSpec((1, 1, blk), lambda i: (i, 0, 0)),
            pl.BlockSpec((_OPF, _OUT), lambda i: (0, 0)),
            pl.BlockSpec((_CFG, _OUT), lambda i: (0, 0)),
            pl.BlockSpec((_EMB, _OUT), lambda i: (0, 0)),
        ],
        out_specs=pl.BlockSpec((blk, _OUT), lambda i: (i, 0)),
        out_shape=jax.ShapeDtypeStruct((n, _OUT), jnp.float32),
        compiler_params=pltpu.CompilerParams(
            dimension_semantics=("parallel",)),
    )(op_feats, config_feats, idx, w_op, w_cfg_f, emb_proj)
    return out


# speedup vs baseline: 1.8056x; 1.0543x over previous
import jax
import jax.numpy as jnp
from jax.experimental import pallas as pl
from jax.experimental.pallas import tpu as pltpu

_OPF = 140
_EMB = 128
_CFG = 18
_OUT = 128


def _prep_kernel(emb_ref, wemb_ref, wcfg_ref, cfgwt_ref, opw_ref, b_ref,
                 proj_ref, wcfgf_ref):
    emb = emb_ref[...]
    norm = jnp.sqrt(jnp.sum(emb * emb, axis=1, keepdims=True))
    scale = jnp.minimum(1.0, 1.0 / jnp.maximum(norm, 1e-7)) * opw_ref[0, 0]
    proj_ref[...] = (
        jnp.dot(emb * scale, wemb_ref[...], preferred_element_type=jnp.float32)
        + b_ref[...]
    )
    wcfgf_ref[...] = wcfg_ref[...] * cfgwt_ref[...]


def _main_kernel(opf_ref, cfg_ref, idx_ref, wop_ref, wcfgf_ref, proj_ref,
                 out_ref):
    blk = opf_ref.shape[0]
    idx = idx_ref[...].reshape(1, blk)
    oh_t = (jax.lax.broadcasted_iota(jnp.int32, (_EMB, blk), 0) == idx
            ).astype(jnp.float32)
    acc = jnp.dot(opf_ref[...], wop_ref[...],
                  preferred_element_type=jnp.float32)
    acc += jnp.dot(cfg_ref[...], wcfgf_ref[...],
                   preferred_element_type=jnp.float32)
    acc += jax.lax.dot_general(oh_t, proj_ref[...],
                               (((0,), (0,)), ((), ())),
                               preferred_element_type=jnp.float32)
    out_ref[...] = acc


def kernel(op_feats, config_feats, emb_table, op_weights, config_weights,
           W, b, op_code):
    n = op_feats.shape[0]
    w_op = W[0:_OPF]
    w_emb = W[_OPF:_OPF + _EMB]
    w_cfg = W[_OPF + _EMB:]
    cfgw_t = config_weights.reshape(_CFG, 1)
    b2 = b.reshape(1, _OUT)

    emb_proj, w_cfg_f = pl.pallas_call(
        _prep_kernel,
        out_shape=[
            jax.ShapeDtypeStruct((_EMB, _OUT), jnp.float32),
            jax.ShapeDtypeStruct((_CFG, _OUT), jnp.float32),
        ],
    )(emb_table, w_emb, w_cfg, cfgw_t, op_weights, b2)

    blk = 5000
    grid = n // blk
    assert grid * blk == n
    idx = op_code.astype(jnp.int32).reshape(grid, 1, blk)

    out = pl.pallas_call(
        _main_kernel,
        grid=(grid,),
        in_specs=[
            pl.BlockSpec((blk, _OPF), lambda i: (i, 0)),
            pl.BlockSpec((blk, _CFG), lambda i: (i, 0)),
            pl.BlockSpec((1, 1, blk), lambda i: (i, 0, 0)),
            pl.BlockSpec((_OPF, _OUT), lambda i: (0, 0)),
            pl.BlockSpec((_CFG, _OUT), lambda i: (0, 0)),
            pl.BlockSpec((_EMB, _OUT), lambda i: (0, 0)),
        ],
        out_specs=pl.BlockSpec((blk, _OUT), lambda i: (i, 0)),
        out_shape=jax.ShapeDtypeStruct((n, _OUT), jnp.float32),
        compiler_params=pltpu.CompilerParams(
            dimension_semantics=("parallel",)),
    )(op_feats, config_feats, idx, w_op, w_cfg_f, emb_proj)
    return out


# speedup vs baseline: 1.8557x; 1.0278x over previous
import jax
import jax.numpy as jnp
from jax.experimental import pallas as pl
from jax.experimental.pallas import tpu as pltpu

_OPF = 140
_EMB = 128
_CFG = 18
_OUT = 128


def _prep_kernel(emb_ref, wemb_ref, wcfg_ref, cfgwt_ref, opw_ref, b_ref,
                 proj_ref, wcfgf_ref):
    emb = emb_ref[...]
    norm = jnp.sqrt(jnp.sum(emb * emb, axis=1, keepdims=True))
    scale = jnp.minimum(1.0, 1.0 / jnp.maximum(norm, 1e-7)) * opw_ref[0, 0]
    proj_ref[...] = (
        jnp.dot(emb * scale, wemb_ref[...], preferred_element_type=jnp.float32)
        + b_ref[...]
    )
    wcfgf_ref[...] = wcfg_ref[...] * cfgwt_ref[...]


def _main_kernel(opf_ref, cfg_ref, idx_ref, wop_ref, wcfgf_ref, proj_ref,
                 out_ref):
    blk = opf_ref.shape[0]
    idx = idx_ref[...].reshape(1, blk)
    oh_t = (jax.lax.broadcasted_iota(jnp.int32, (_EMB, blk), 0) == idx
            ).astype(jnp.float32)
    acc = jnp.dot(opf_ref[...], wop_ref[...],
                  preferred_element_type=jnp.float32)
    acc += jnp.dot(cfg_ref[...], wcfgf_ref[...],
                   preferred_element_type=jnp.float32)
    acc += jax.lax.dot_general(oh_t, proj_ref[...],
                               (((0,), (0,)), ((), ())),
                               preferred_element_type=jnp.float32)
    out_ref[...] = acc


def kernel(op_feats, config_feats, emb_table, op_weights, config_weights,
           W, b, op_code):
    n = op_feats.shape[0]
    w_op = W[0:_OPF]
    w_emb = W[_OPF:_OPF + _EMB]
    w_cfg = W[_OPF + _EMB:]
    cfgw_t = config_weights.reshape(_CFG, 1)
    b2 = b.reshape(1, _OUT)

    emb_proj, w_cfg_f = pl.pallas_call(
        _prep_kernel,
        out_shape=[
            jax.ShapeDtypeStruct((_EMB, _OUT), jnp.float32),
            jax.ShapeDtypeStruct((_CFG, _OUT), jnp.float32),
        ],
    )(emb_table, w_emb, w_cfg, cfgw_t, op_weights, b2)

    blk = 10000
    grid = n // blk
    assert grid * blk == n
    idx = op_code.astype(jnp.int32).reshape(grid, 1, blk)

    out = pl.pallas_call(
        _main_kernel,
        grid=(grid,),
        in_specs=[
            pl.BlockSpec((blk, _OPF), lambda i: (i, 0)),
            pl.BlockSpec((blk, _CFG), lambda i: (i, 0)),
            pl.BlockSpec((1, 1, blk), lambda i: (i, 0, 0)),
            pl.BlockSpec((_OPF, _OUT), lambda i: (0, 0)),
            pl.BlockSpec((_CFG, _OUT), lambda i: (0, 0)),
            pl.BlockSpec((_EMB, _OUT), lambda i: (0, 0)),
        ],
        out_specs=pl.BlockSpec((blk, _OUT), lambda i: (i, 0)),
        out_shape=jax.ShapeDtypeStruct((n, _OUT), jnp.float32),
        compiler_params=pltpu.CompilerParams(
            dimension_semantics=("parallel",)),
    )(op_feats, config_feats, idx, w_op, w_cfg_f, emb_proj)
    return out
